# baseline (speedup 1.0000x reference)
import functools

import jax
import jax.numpy as jnp
from jax.experimental import pallas as pl

_HF = 64


def _mm_body(x_ref, w_ref, b_ref, o_ref):
    o_ref[...] = jnp.dot(x_ref[...], w_ref[...],
                         preferred_element_type=jnp.float32) + b_ref[...]


def _pallas_mm(x, W, b, blk_rows=256):
    n, k = x.shape
    ko = W.shape[1]
    n_pad = ((n + blk_rows - 1) // blk_rows) * blk_rows
    k_pad = ((k + 127) // 128) * 128
    ko_pad = ((ko + 127) // 128) * 128
    xp = jnp.zeros((n_pad, k_pad), x.dtype).at[:n, :k].set(x)
    Wp = jnp.zeros((k_pad, ko_pad), W.dtype).at[:k, :ko].set(W)
    bp = jnp.zeros((1, ko_pad), b.dtype).at[0, :ko].set(b)
    out = pl.pallas_call(
        _mm_body,
        grid=(n_pad // blk_rows,),
        in_specs=[
            pl.BlockSpec((blk_rows, k_pad), lambda i: (i, 0)),
            pl.BlockSpec((k_pad, ko_pad), lambda i: (0, 0)),
            pl.BlockSpec((1, ko_pad), lambda i: (0, 0)),
        ],
        out_specs=pl.BlockSpec((blk_rows, ko_pad), lambda i: (i, 0)),
        out_shape=jax.ShapeDtypeStruct((n_pad, ko_pad), jnp.float32),
    )(xp, Wp, bp)
    return out[:n, :ko]


def _bnorm(h, g, b):
    m = jnp.mean(h, axis=0, keepdims=True)
    v = jnp.var(h, axis=0, keepdims=True)
    return (h - m) / jnp.sqrt(v + 1e-5) * g + b


def _gat_layer(h, W, al, ar, b, src, dst, n, heads):
    feat = (h @ W).reshape(n, heads, _HF)
    el = jnp.sum(feat * al[None, :, :], axis=-1)
    er = jnp.sum(feat * ar[None, :, :], axis=-1)
    e = jax.nn.leaky_relu(el[src] + er[dst], 0.2)
    emax = jax.ops.segment_max(e, dst, num_segments=n)
    emax = jnp.where(jnp.isfinite(emax), emax, 0.0)
    ee = jnp.exp(e - emax[dst])
    den = jax.ops.segment_sum(ee, dst, num_segments=n)
    alpha = ee / (den[dst] + 1e-9)
    msg = feat[src] * alpha[:, :, None]
    rst = jax.ops.segment_sum(msg, dst, num_segments=n)
    return rst + b.reshape(1, heads, _HF)


def kernel(x, params, edge_index):
    src = edge_index[0]
    dst = edge_index[1]
    n = x.shape[0]
    h = _pallas_mm(x, params['enc_W0'], params['enc_b0'])
    h = jax.nn.leaky_relu(_bnorm(h, params['bn_g0'], params['bn_b0']), 0.01)
    h = _pallas_mm(h, params['enc_W1'], params['enc_b1'])
    h = jax.nn.leaky_relu(_bnorm(h, params['bn_g1'], params['bn_b1']), 0.01)
    heads_per_layer = [4, 4, 4, 4, 1]
    for i in range(5):
        hd = heads_per_layer[i]
        h = _gat_layer(h, params['gat_W'][i], params['gat_al'][i],
                       params['gat_ar'][i], params['gat_b'][i], src, dst, n, hd)
        h = jax.nn.leaky_relu(h, 0.01).reshape(n, hd * _HF)
    out = _pallas_mm(h, params['fc_W'], params['fc_b'])
    return out


# speedup vs baseline: 15.5363x; 15.5363x over previous
import functools

import jax
import jax.numpy as jnp
from jax import lax
from jax.experimental import pallas as pl
from jax.experimental.pallas import tpu as pltpu
from jax.experimental.pallas import tpu_sc as plsc

_N = 50000
_E = 800000
_HF = 64

_TILES = 32
_NBLK = 13
_ND_B = 128
_N_PAD = _TILES * _NBLK * _ND_B
_C = 128
_E_PAD = _E + 2 * _C
_RB = 256
_NGRID = _N_PAD // _RB


def _enc0_body(x_ref, w_ref, b_ref, o_ref, st_ref):
    i = pl.program_id(0)
    h = jnp.dot(x_ref[...], w_ref[...], preferred_element_type=jnp.float32)
    h = h + b_ref[0:1, :]
    o_ref[...] = h
    rows = i * _RB + lax.broadcasted_iota(jnp.int32, (_RB, 1), 0)
    hm = jnp.where(rows < _N, h, 0.0)

    @pl.when(i == 0)
    def _():
        st_ref[...] = jnp.zeros_like(st_ref)

    st_ref[0:1, :] += jnp.sum(hm, axis=0, keepdims=True)
    st_ref[1:2, :] += jnp.sum(hm * hm, axis=0, keepdims=True)


def _bn_mm_body(h_ref, st_ref, bn_ref, w_ref, b_ref, o_ref, st2_ref):
    i = pl.program_id(0)
    m = st_ref[0:1, :] / _N
    v = st_ref[1:2, :] / _N - m * m
    a = (h_ref[...] - m) * lax.rsqrt(v + 1e-5) * bn_ref[0:1, :] + bn_ref[1:2, :]
    a = jnp.where(a > 0, a, 0.01 * a)
    h = jnp.dot(a, w_ref[...], preferred_element_type=jnp.float32)
    h = h + b_ref[0:1, :]
    o_ref[...] = h
    rows = i * _RB + lax.broadcasted_iota(jnp.int32, (_RB, 1), 0)
    hm = jnp.where(rows < _N, h, 0.0)

    @pl.when(i == 0)
    def _():
        st2_ref[...] = jnp.zeros_like(st2_ref)

    st2_ref[0:1, :] += jnp.sum(hm, axis=0, keepdims=True)
    st2_ref[1:2, :] += jnp.sum(hm * hm, axis=0, keepdims=True)


def _bn_feat_body(h_ref, st_ref, bn_ref, w_ref, f_ref):
    m = st_ref[0:1, :] / _N
    v = st_ref[1:2, :] / _N - m * m
    a = (h_ref[...] - m) * lax.rsqrt(v + 1e-5) * bn_ref[0:1, :] + bn_ref[1:2, :]
    a = jnp.where(a > 0, a, 0.01 * a)
    f_ref[...] = jnp.dot(a, w_ref[...], preferred_element_type=jnp.float32)


def _gat_act(acc_ref, ws_ref, b_ref, H):
    FP = acc_ref.shape[1]
    w = ws_ref[...]
    parts = []
    for g in range(FP // _HF):
        hh = min(g, H - 1)
        parts.append(jnp.broadcast_to(w[:, hh:hh + 1], (_RB, _HF)))
    den = jnp.concatenate(parts, axis=1) + 1e-9
    a = acc_ref[...] / den + b_ref[0:1, :]
    return jnp.where(a > 0, a, 0.01 * a)


def _gat_feat_body(acc_ref, ws_ref, b_ref, w_ref, f_ref, *, H):
    a = _gat_act(acc_ref, ws_ref, b_ref, H)
    f_ref[...] = jnp.dot(a, w_ref[...], preferred_element_type=jnp.float32)


def _gat_fc_body(acc_ref, ws_ref, b_ref, w_ref, fcb_ref, o_ref, *, H):
    a = _gat_act(acc_ref, ws_ref, b_ref, H)
    o_ref[...] = jnp.dot(a, w_ref[...],
                         preferred_element_type=jnp.float32) + fcb_ref[0:1, :]


def _row8(vec, width):
    out = jnp.zeros((8, width), jnp.float32)
    return out.at[0, :vec.shape[0]].set(vec)


def _bn8(g, b, width):
    out = jnp.zeros((8, width), jnp.float32)
    return out.at[0, :g.shape[0]].set(g).at[1, :b.shape[0]].set(b)


def _enc0(xp, W, b):
    return pl.pallas_call(
        _enc0_body,
        grid=(_NGRID,),
        in_specs=[
            pl.BlockSpec((_RB, 64), lambda i: (i, 0)),
            pl.BlockSpec((64, 64), lambda i: (0, 0)),
            pl.BlockSpec((8, 64), lambda i: (0, 0)),
        ],
        out_specs=[
            pl.BlockSpec((_RB, 64), lambda i: (i, 0)),
            pl.BlockSpec((8, 64), lambda i: (0, 0)),
        ],
        out_shape=[
            jax.ShapeDtypeStruct((_N_PAD, 64), jnp.float32),
            jax.ShapeDtypeStruct((8, 64), jnp.float32),
        ],
    )(xp, W, b)


def _bn_mm(h, st, bn, W, b):
    return pl.pallas_call(
        _bn_mm_body,
        grid=(_NGRID,),
        in_specs=[
            pl.BlockSpec((_RB, 64), lambda i: (i, 0)),
            pl.BlockSpec((8, 64), lambda i: (0, 0)),
            pl.BlockSpec((8, 64), lambda i: (0, 0)),
            pl.BlockSpec((64, 64), lambda i: (0, 0)),
            pl.BlockSpec((8, 64), lambda i: (0, 0)),
        ],
        out_specs=[
            pl.BlockSpec((_RB, 64), lambda i: (i, 0)),
            pl.BlockSpec((8, 64), lambda i: (0, 0)),
        ],
        out_shape=[
            jax.ShapeDtypeStruct((_N_PAD, 64), jnp.float32),
            jax.ShapeDtypeStruct((8, 64), jnp.float32),
        ],
    )(h, st, bn, W, b)


def _bn_feat(h, st, bn, Wcat):
    Wt = Wcat.shape[1]
    return pl.pallas_call(
        _bn_feat_body,
        grid=(_NGRID,),
        in_specs=[
            pl.BlockSpec((_RB, 64), lambda i: (i, 0)),
            pl.BlockSpec((8, 64), lambda i: (0, 0)),
            pl.BlockSpec((8, 64), lambda i: (0, 0)),
            pl.BlockSpec((64, Wt), lambda i: (0, 0)),
        ],
        out_specs=pl.BlockSpec((_RB, Wt), lambda i: (i, 0)),
        out_shape=jax.ShapeDtypeStruct((_N_PAD, Wt), jnp.float32),
    )(h, st, bn, Wcat)


def _gat_feat_mm(acc, ws, bias, Wcat, H):
    din = acc.shape[1]
    Wt = Wcat.shape[1]
    return pl.pallas_call(
        functools.partial(_gat_feat_body, H=H),
        grid=(_NGRID,),
        in_specs=[
            pl.BlockSpec((_RB, din), lambda i: (i, 0)),
            pl.BlockSpec((_RB, 16), lambda i: (i, 0)),
            pl.BlockSpec((8, din), lambda i: (0, 0)),
            pl.BlockSpec((din, Wt), lambda i: (0, 0)),
        ],
        out_specs=pl.BlockSpec((_RB, Wt), lambda i: (i, 0)),
        out_shape=jax.ShapeDtypeStruct((_N_PAD, Wt), jnp.float32),
    )(acc, ws, bias, Wcat)


def _gat_fc(acc, ws, bias, W, fcb, H):
    return pl.pallas_call(
        functools.partial(_gat_fc_body, H=H),
        grid=(_NGRID,),
        in_specs=[
            pl.BlockSpec((_RB, 128), lambda i: (i, 0)),
            pl.BlockSpec((_RB, 16), lambda i: (i, 0)),
            pl.BlockSpec((8, 128), lambda i: (0, 0)),
            pl.BlockSpec((128, 128), lambda i: (0, 0)),
            pl.BlockSpec((8, 128), lambda i: (0, 0)),
        ],
        out_specs=pl.BlockSpec((_RB, 128), lambda i: (i, 0)),
        out_shape=jax.ShapeDtypeStruct((_N_PAD, 128), jnp.float32),
    )(acc, ws, bias, W, fcb)


@functools.lru_cache(maxsize=None)
def _make_sc_edge(F, H):
    FP = ((F + 127) // 128) * 128
    W = FP + 128
    NJ = F // 16
    NJF = FP // 16
    mesh = plsc.VectorSubcoreMesh(core_axis_name="c", subcore_axis_name="s")

    @functools.partial(
        pl.kernel,
        mesh=mesh,
        compiler_params=pltpu.CompilerParams(needs_layout_passes=False),
        out_type=[
            jax.ShapeDtypeStruct((_N_PAD, FP), jnp.float32),
            jax.ShapeDtypeStruct((_N_PAD, 16), jnp.float32),
        ],
        scratch_types=[
            pltpu.VMEM((_C,), jnp.int32),
            pltpu.VMEM((_C // 16, 16), jnp.int32),
            pltpu.VMEM((_C, W), jnp.float32),
            pltpu.VMEM((_ND_B, 128), jnp.float32),
            pltpu.VMEM((_ND_B, FP), jnp.float32),
            pltpu.VMEM((_ND_B, 16), jnp.float32),
            pltpu.VMEM((_TILES * 16,), jnp.int32),
            pltpu.SemaphoreType.DMA,
        ],
    )
    def sc_edge(fe_hbm, srcs_hbm, dsts_hbm, bp2_hbm,
                out_hbm, ws_hbm, idx_v, dst_v, rows_v, er_v, acc_v, wsum_v,
                bp_v, sem0):
        wid = lax.axis_index("s") * 2 + lax.axis_index("c")
        pltpu.sync_copy(bp2_hbm, bp_v)
        zvec = jnp.zeros((16,), jnp.float32)

        def block_body(blk, _):
            d0 = pl.multiple_of((wid * _NBLK + blk) * _ND_B, _ND_B)
            bsel = jnp.full((16,), wid * 16 + blk, jnp.int32)
            e_lo = plsc.load_gather(bp_v, [bsel])[0]
            e_hi = plsc.load_gather(bp_v, [bsel + 1])[0]

            def zero_body(d, _):
                for j in range(NJF):
                    acc_v[d, pl.ds(16 * j, 16)] = zvec
                wsum_v[d, :] = zvec
                return 0

            lax.fori_loop(0, _ND_B, zero_body, 0)
            pltpu.sync_copy(fe_hbm.at[pl.ds(d0, _ND_B), pl.ds(FP, 128)], er_v)

            a_lo = e_lo & ~(_C - 1)
            nchunks = (e_hi - a_lo + _C - 1) // _C

            def chunk_body(c, _, e_lo=e_lo, e_hi=e_hi, a_lo=a_lo, d0=d0):
                a = pl.multiple_of(a_lo + c * _C, _C)
                pltpu.sync_copy(srcs_hbm.at[pl.ds(a, _C)], idx_v)
                pltpu.sync_copy(
                    dsts_hbm.at[pl.ds(pl.multiple_of(a // 16, 8), _C // 16)],
                    dst_v)
                pltpu.async_copy(fe_hbm.at[idx_v], rows_v, sem0).wait()

                def grp_body(g, _, d0=d0):
                    dvec = dst_v[g, :]
                    for i in range(16):
                        k = g * 16 + i
                        kg = a + k
                        valid = (kg >= e_lo) & (kg < e_hi)
                        d_l = jnp.clip(dvec[i] - d0, 0, _ND_B - 1)
                        sel = jnp.where(valid, 1.0, 0.0)
                        el = rows_v[k, pl.ds(FP, 16)]
                        er = er_v[d_l, pl.ds(16, 16)]
                        e = el + er
                        e = jnp.where(e > 0, e, 0.2 * e)
                        ee = jnp.exp(e) * sel
                        wsum_v[d_l, :] = wsum_v[d_l, :] + ee
                        sc = [ee[h] for h in range(H)]
                        for j in range(NJ):
                            val = (rows_v[k, pl.ds(16 * j, 16)]
                                   * sc[(16 * j) // _HF])
                            acc_v[d_l, pl.ds(16 * j, 16)] = (
                                acc_v[d_l, pl.ds(16 * j, 16)] + val)
                    return 0

                lax.fori_loop(0, _C // 16, grp_body, 0)
                return 0

            lax.fori_loop(0, nchunks, chunk_body, 0)

            pltpu.sync_copy(acc_v, out_hbm.at[pl.ds(d0, _ND_B)])
            pltpu.sync_copy(wsum_v, ws_hbm.at[pl.ds(d0, _ND_B)])
            return 0

        lax.fori_loop(0, _NBLK, block_body, 0)

    return sc_edge


def _attn_weights(W, al, ar, H):
    din = W.shape[0]
    F = H * _HF
    FP = ((F + 127) // 128) * 128
    Wl = jnp.einsum('dhf,hf->dh', W.reshape(din, H, _HF), al)
    Wr = jnp.einsum('dhf,hf->dh', W.reshape(din, H, _HF), ar)
    Wcat = jnp.zeros((din, FP + 128), jnp.float32)
    Wcat = Wcat.at[:, :F].set(W)
    Wcat = Wcat.at[:, FP:FP + H].set(Wl)
    Wcat = Wcat.at[:, FP + 16:FP + 16 + H].set(Wr)
    return Wcat


def kernel(x, params, edge_index):
    src = edge_index[0].astype(jnp.int32)
    dst = edge_index[1].astype(jnp.int32)

    dsts, srcs = lax.sort_key_val(dst, src)
    bounds = jnp.arange(_TILES * _NBLK + 1, dtype=jnp.int32) * _ND_B
    bptr = jnp.searchsorted(dsts, bounds).astype(jnp.int32)
    bidx = (jnp.arange(_TILES)[:, None] * _NBLK
            + jnp.arange(16)[None, :]).clip(0, _TILES * _NBLK)
    bp2 = bptr[bidx].reshape(-1)
    srcs_p = jnp.zeros((_E_PAD,), jnp.int32).at[:_E].set(srcs)
    dsts_p = jnp.zeros((_E_PAD,), jnp.int32).at[:_E].set(dsts)
    dsts_p = dsts_p.reshape(_E_PAD // 16, 16)

    xp = jnp.zeros((_N_PAD, 64), jnp.float32).at[:_N, :33].set(x)

    h0, st0 = _enc0(xp, jnp.zeros((64, 64), jnp.float32).at[:33, :].set(
        params['enc_W0']), _row8(params['enc_b0'], 64))
    h1, st1 = _bn_mm(h0, st0, _bn8(params['bn_g0'], params['bn_b0'], 64),
                     params['enc_W1'], _row8(params['enc_b1'], 64))

    heads = [4, 4, 4, 4, 1]
    wcats = [_attn_weights(params['gat_W'][i], params['gat_al'][i],
                           params['gat_ar'][i], heads[i]) for i in range(5)]

    fe = _bn_feat(h1, st1, _bn8(params['bn_g1'], params['bn_b1'], 64),
                  wcats[0])
    out = None
    for i in range(5):
        H = heads[i]
        F = H * _HF
        FP = ((F + 127) // 128) * 128
        acc, ws = _make_sc_edge(F, H)(fe, srcs_p, dsts_p, bp2)
        bias_p = _row8(params['gat_b'][i], FP)
        if i < 4:
            fe = _gat_feat_mm(acc, ws, bias_p, wcats[i + 1], H)
        else:
            fcW = jnp.zeros((128, 128), jnp.float32).at[:64, :2].set(
                params['fc_W'])
            out = _gat_fc(acc, ws, bias_p, fcW, _row8(params['fc_b'], 128), H)
    return out[:_N, :2]
